# baseline (speedup 1.0000x reference)
import jax
import jax.numpy as jnp
from jax import lax
from jax.experimental import pallas as pl
from jax.experimental.pallas import tpu as pltpu
from jax.experimental.pallas import tpu_sc as plsc

NC = 2
NS = 16
L = 16
NW = NC * NS
B = 16384
D = 16
BPW = B // NW
CH = 128
NCHUNK = BPW // CH


def _irt_body(sid_hbm, qid_hbm, theta_hbm, alpha_hbm, beta_hbm, out_hbm,
              sidx, qidx, th_v, al_v, be_v, out_v, sem):
    wid = lax.axis_index("s") * NC + lax.axis_index("c")
    base = wid * BPW
    row0 = wid * NCHUNK

    pltpu.sync_copy(sid_hbm.at[pl.ds(row0, NCHUNK)], sidx)
    pltpu.sync_copy(qid_hbm.at[pl.ds(row0, NCHUNK)], qidx)

    copies = []
    for j in range(NCHUNK):
        dst = pl.ds(j * CH, CH)
        copies.append(pltpu.async_copy(theta_hbm.at[sidx.at[j]], th_v.at[dst], sem))
        copies.append(pltpu.async_copy(alpha_hbm.at[qidx.at[j]], al_v.at[dst], sem))
        copies.append(pltpu.async_copy(beta_hbm.at[qidx.at[j]], be_v.at[dst], sem))
    for c in copies:
        c.wait()

    def blk_body(blk, carry):
        r = blk * L
        rows = r + lax.iota(jnp.int32, L)
        acc = be_v[pl.ds(r, L)]
        for d in range(D):
            cols = jnp.full((L,), d, jnp.int32)
            t = plsc.load_gather(th_v, [rows, cols])
            a = plsc.load_gather(al_v, [rows, cols])
            acc = acc + t * a
        out_v[pl.ds(r, L)] = 1.0 / (1.0 + jnp.exp(-acc))
        return carry

    lax.fori_loop(0, BPW // L, blk_body, jnp.int32(0))
    pltpu.sync_copy(out_v, out_hbm.at[pl.ds(base, BPW)])


@jax.jit
def kernel(student_ids, question_ids, theta_w, alpha_w, beta_w):
    sid2 = student_ids.astype(jnp.int32).reshape(NW * NCHUNK, CH)
    qid2 = question_ids.astype(jnp.int32).reshape(NW * NCHUNK, CH)
    beta1 = beta_w.reshape(-1)
    run = pl.kernel(
        _irt_body,
        out_type=jax.ShapeDtypeStruct((B,), jnp.float32),
        mesh=plsc.VectorSubcoreMesh(core_axis_name="c", subcore_axis_name="s"),
        scratch_types=[
            pltpu.VMEM((NCHUNK, CH), jnp.int32),
            pltpu.VMEM((NCHUNK, CH), jnp.int32),
            pltpu.VMEM((BPW, D), jnp.float32),
            pltpu.VMEM((BPW, D), jnp.float32),
            pltpu.VMEM((BPW,), jnp.float32),
            pltpu.VMEM((BPW,), jnp.float32),
            pltpu.SemaphoreType.DMA,
        ],
        compiler_params=pltpu.CompilerParams(
            needs_layout_passes=False, use_tc_tiling_on_sc=False),
    )
    out = run(sid2, qid2, theta_w, alpha_w, beta1)
    return out.reshape(B, 1)
